# baseline (speedup 1.0000x reference)
import functools

import jax
import jax.numpy as jnp
from jax import lax
from jax.experimental import pallas as pl
from jax.experimental.pallas import tpu as pltpu
from jax.experimental.pallas import tpu_sc as plsc

_NUM_GROUPS = 16
_FEAT = 256
_FH = 64
_FW = 64
_HW = _FH * _FW
_ROWS = _NUM_GROUPS * _HW

_NC = 2
_NS = 16
_NW = _NC * _NS
_CHUNK = 128


def _transpose_body(in_ref, out_ref):
    out_ref[0] = in_ref[0].T


def _build_table(feat_map):
    fm3 = feat_map.reshape(_NUM_GROUPS, _FEAT, _HW)
    t = pl.pallas_call(
        _transpose_body,
        grid=(_NUM_GROUPS, 8),
        in_specs=[pl.BlockSpec((1, _FEAT, _HW // 8), lambda g, j: (g, 0, j))],
        out_specs=pl.BlockSpec((1, _HW // 8, _FEAT), lambda g, j: (g, j, 0)),
        out_shape=jax.ShapeDtypeStruct((_NUM_GROUPS, _HW, _FEAT), jnp.float32),
    )(fm3)
    return t.reshape(_ROWS, _FEAT)


def _gather(table, grp, grid, n_pad):
    chunks_per_w = n_pad // (_NW * _CHUNK)
    assert chunks_per_w % 2 == 0 and chunks_per_w >= 4
    rows_per_w = chunks_per_w * _CHUNK
    mesh = plsc.VectorSubcoreMesh(core_axis_name="c", subcore_axis_name="s")

    @functools.partial(
        pl.kernel,
        mesh=mesh,
        compiler_params=pltpu.CompilerParams(needs_layout_passes=False),
        out_type=jax.ShapeDtypeStruct((n_pad, _FEAT), jnp.float32),
        scratch_types=[
            pltpu.VMEM((_CHUNK,), jnp.int32),
            pltpu.VMEM((_CHUNK,), jnp.int32),
            pltpu.VMEM((2 * _CHUNK,), jnp.int32),
            pltpu.VMEM((2 * _CHUNK,), jnp.int32),
            pltpu.VMEM((_CHUNK,), jnp.int32),
            pltpu.VMEM((_CHUNK,), jnp.int32),
            pltpu.VMEM((_CHUNK, _FEAT), jnp.float32),
            pltpu.VMEM((_CHUNK, _FEAT), jnp.float32),
            pltpu.SemaphoreType.DMA,
            pltpu.SemaphoreType.DMA,
            pltpu.SemaphoreType.DMA,
            pltpu.SemaphoreType.DMA,
            pltpu.SemaphoreType.DMA,
            pltpu.SemaphoreType.DMA,
        ],
    )
    def k(table_hbm, grp_hbm, grid_hbm, out_hbm,
          grp0, grp1, grid0, grid1, idx0, idx1, rows0, rows1,
          ssem0, ssem1, gsem0, gsem1, wsem0, wsem1):
        wid = lax.axis_index("s") * _NC + lax.axis_index("c")
        w_base = wid * rows_per_w
        w_last = w_base + rows_per_w - _CHUNK

        grp_b = (grp0, grp1)
        grid_b = (grid0, grid1)
        idx_b = (idx0, idx1)
        rows_b = (rows0, rows1)
        ssem_b = (ssem0, ssem1)
        gsem_b = (gsem0, gsem1)
        wsem_b = (wsem0, wsem1)

        def start_slabs(j, b):
            base = jnp.minimum(w_base + j * _CHUNK, w_last)
            pltpu.async_copy(grp_hbm.at[pl.ds(base, _CHUNK)], grp_b[b], ssem_b[b])
            pltpu.async_copy(
                grid_hbm.at[pl.ds(2 * base, 2 * _CHUNK)], grid_b[b], ssem_b[b])

        def wait_slabs(b):
            pltpu.make_async_copy(
                grp_hbm.at[pl.ds(0, _CHUNK)], grp_b[b], ssem_b[b]).wait()
            pltpu.make_async_copy(
                grid_hbm.at[pl.ds(0, 2 * _CHUNK)], grid_b[b], ssem_b[b]).wait()

        def compute_idx(b):
            for i in range(_CHUNK // 16):
                pairs = (lax.iota(jnp.int32, 16) + jnp.int32(i * 16)) * 2
                g = grp_b[b][pl.ds(i * 16, 16)]
                x = plsc.load_gather(grid_b[b], [pairs])
                y = plsc.load_gather(grid_b[b], [pairs + 1])
                idx_b[b][pl.ds(i * 16, 16)] = g * _HW + y * _FW + x

        def wait_write(b):
            pltpu.make_async_copy(
                rows_b[b], out_hbm.at[pl.ds(w_base, _CHUNK)], wsem_b[b]).wait()

        def chunk(j, b, first):
            wait_slabs(b)
            compute_idx(b)
            start_slabs(j + 2, b)
            if not first:
                wait_write(b)
            g = pltpu.async_copy(table_hbm.at[idx_b[b]], rows_b[b], gsem_b[b])
            g.wait()
            base = w_base + j * _CHUNK
            pltpu.async_copy(rows_b[b], out_hbm.at[pl.ds(base, _CHUNK)], wsem_b[b])

        start_slabs(jnp.int32(0), 0)
        start_slabs(jnp.int32(1), 1)
        chunk(jnp.int32(0), 0, True)
        chunk(jnp.int32(1), 1, True)

        def body(jj, carry):
            j = jj * 2
            chunk(j, 0, False)
            chunk(j + 1, 1, False)
            return carry

        lax.fori_loop(1, chunks_per_w // 2, body, 0)

        wait_write(0)
        wait_write(1)
        wait_slabs(0)
        wait_slabs(1)

    return k(table, grp, grid)


def kernel(feat_map, grp_ids, grid_ids):
    n = grp_ids.shape[0]
    grp = grp_ids.astype(jnp.int32)
    grid = grid_ids.astype(jnp.int32)
    per = _NW * _CHUNK * 2
    n_pad = ((n + per - 1) // per) * per
    pad = n_pad - n
    if pad:
        grp = jnp.pad(grp, (0, pad))
        grid = jnp.pad(grid, ((0, pad), (0, 0)))
    grid = grid.reshape(-1)
    table = _build_table(feat_map)
    out = _gather(table, grp, grid, n_pad)
    return out[:n]


# speedup vs baseline: 5.6470x; 5.6470x over previous
import functools

import jax
import jax.numpy as jnp
from jax import lax
from jax.experimental import pallas as pl
from jax.experimental.pallas import tpu as pltpu
from jax.experimental.pallas import tpu_sc as plsc

_NUM_GROUPS = 16
_FEAT = 256
_FH = 64
_FW = 64
_HW = _FH * _FW
_ROWS = _NUM_GROUPS * _HW

_NC = 2
_NS = 16
_NW = _NC * _NS
_CHUNK = 128


def _gather(table, grp, xs, ys, n):
    n_chunks = -(-n // _CHUNK)
    last_chunk = n_chunks - 1
    last_base = n - _CHUNK
    chunks_per_w = -(-n_chunks // _NW)
    assert chunks_per_w % 2 == 1 and chunks_per_w >= 5
    mesh = plsc.VectorSubcoreMesh(core_axis_name="c", subcore_axis_name="s")

    @functools.partial(
        pl.kernel,
        mesh=mesh,
        compiler_params=pltpu.CompilerParams(needs_layout_passes=False),
        out_type=jax.ShapeDtypeStruct((n, _FEAT), jnp.float32),
        scratch_types=(
            [pltpu.VMEM((_CHUNK,), jnp.int32) for _ in range(2)]
            + [pltpu.VMEM((_CHUNK,), jnp.int32) for _ in range(2)]
            + [pltpu.VMEM((_CHUNK,), jnp.int32) for _ in range(2)]
            + [pltpu.VMEM((_CHUNK,), jnp.int32) for _ in range(2)]
            + [pltpu.VMEM((_CHUNK, _FEAT), jnp.float32) for _ in range(2)]
            + [pltpu.SemaphoreType.DMA for _ in range(6)]
        ),
    )
    def k(table_hbm, grp_hbm, xs_hbm, ys_hbm, out_hbm, *scratch):
        grp_b = scratch[0:2]
        xs_b = scratch[2:4]
        ys_b = scratch[4:6]
        idx_b = scratch[6:8]
        rows_b = scratch[8:10]
        ssem_b = scratch[10:12]
        gsem_b = scratch[12:14]
        wsem_b = scratch[14:16]

        wid = lax.axis_index("s") * _NC + lax.axis_index("c")

        def chunk_base(j):
            t = jnp.minimum(wid + j * _NW, last_chunk)
            return jnp.minimum(t * _CHUNK, last_base)

        def start_slabs(j, b):
            base = chunk_base(j)
            pltpu.async_copy(grp_hbm.at[pl.ds(base, _CHUNK)], grp_b[b], ssem_b[b])
            pltpu.async_copy(xs_hbm.at[pl.ds(base, _CHUNK)], xs_b[b], ssem_b[b])
            pltpu.async_copy(ys_hbm.at[pl.ds(base, _CHUNK)], ys_b[b], ssem_b[b])

        def wait_slabs(b):
            for ref in (grp_b[b], xs_b[b], ys_b[b]):
                pltpu.make_async_copy(
                    grp_hbm.at[pl.ds(0, _CHUNK)], ref, ssem_b[b]).wait()

        def compute_idx(b):
            for i in range(_CHUNK // 16):
                sl = pl.ds(i * 16, 16)
                g = grp_b[b][sl]
                x = xs_b[b][sl]
                y = ys_b[b][sl]
                idx_b[b][sl] = g * _HW + y * _FW + x

        def wait_write(b):
            pltpu.make_async_copy(
                rows_b[b], out_hbm.at[pl.ds(0, _CHUNK)], wsem_b[b]).wait()

        def wait_gather(b):
            pltpu.make_async_copy(
                table_hbm.at[idx_b[b]], rows_b[b], gsem_b[b]).wait()

        def chunk(j, b, ring_warm, have_prev):
            wait_slabs(b)
            compute_idx(b)
            start_slabs(j + 2, b)
            if ring_warm:
                wait_write(b)
            pltpu.async_copy(table_hbm.at[idx_b[b]], rows_b[b], gsem_b[b])
            if have_prev:
                pb = 1 - b
                wait_gather(pb)
                pltpu.async_copy(
                    rows_b[pb], out_hbm.at[pl.ds(chunk_base(j - 1), _CHUNK)],
                    wsem_b[pb])

        start_slabs(jnp.int32(0), 0)
        start_slabs(jnp.int32(1), 1)
        chunk(jnp.int32(0), 0, False, False)
        chunk(jnp.int32(1), 1, False, True)

        def body(jj, carry):
            j = jj * 2
            chunk(j, 0, True, True)
            chunk(j + 1, 1, True, True)
            return carry

        lax.fori_loop(1, chunks_per_w // 2, body, 0)

        chunk(jnp.int32(chunks_per_w - 1), 0, True, True)
        lb = 0
        wait_gather(lb)
        pltpu.async_copy(
            rows_b[lb],
            out_hbm.at[pl.ds(chunk_base(jnp.int32(chunks_per_w - 1)), _CHUNK)],
            wsem_b[lb])
        for b in range(2):
            wait_write(b)
            wait_slabs(b)

    return k(table, grp, xs, ys)


def kernel(feat_map, grp_ids, grid_ids):
    n = grp_ids.shape[0]
    table = jnp.transpose(feat_map, (0, 2, 3, 1)).reshape(_ROWS, _FEAT)
    grp = grp_ids.astype(jnp.int32)
    xs = grid_ids[:, 0].astype(jnp.int32)
    ys = grid_ids[:, 1].astype(jnp.int32)
    return _gather(table, grp, xs, ys, n)
